# baseline (speedup 1.0000x reference)
import functools

import jax
import jax.numpy as jnp
from jax import lax
from jax.experimental import pallas as pl
from jax.experimental.pallas import tpu as pltpu
from jax.experimental.pallas import tpu_sc as plsc

N = 10000
HID = 256
HHALF = 128
E = 160000
NSEG = 64
L = 3
OUT = 64
BN_EPS = 1e-05

NPAD = 10240
EPAD = 163840
ROWBLK = 512
GRID = NPAD // ROWBLK
CHUNK = 128
EPG = EPAD // 16
NCH = EPG // CHUNK
RPT = NPAD // 16


def _make_agg():
    mesh = plsc.VectorSubcoreMesh(core_axis_name="c", subcore_axis_name="s")
    MEGA = NCH // 4

    @functools.partial(
        pl.kernel,
        out_type=[
            jax.ShapeDtypeStruct((NPAD, HHALF), jnp.float32),
            jax.ShapeDtypeStruct((NPAD, HHALF), jnp.float32),
        ],
        mesh=mesh,
        scratch_types=[
            [[pltpu.VMEM((CHUNK,), jnp.int32) for _ in range(2)] for _ in range(2)],
            [[pltpu.VMEM((CHUNK,), jnp.int32) for _ in range(2)] for _ in range(2)],
            [pltpu.VMEM((CHUNK, HHALF), jnp.float32) for _ in range(2)],
            [pltpu.SemaphoreType.DMA for _ in range(2)],
            [[pltpu.SemaphoreType.DMA for _ in range(2)] for _ in range(2)],
            pltpu.VMEM_SHARED((NPAD, HHALF), jnp.float32),
        ],
    )
    def agg_kernel(h0, h1, src, dst, zrows, out0, out1,
                   sidx, didx, rows, gsem, ssem, shared):
        c = lax.axis_index("c")
        s = lax.axis_index("s")
        pltpu.sync_copy(zrows, shared.at[pl.ds(s * RPT, RPT)])
        plsc.subcore_barrier()
        ebase = s * EPG

        def load_idx(slot, b, chunk):
            pltpu.sync_copy(src.at[pl.ds(ebase + chunk * CHUNK, CHUNK)],
                            sidx[slot][b])
            pltpu.sync_copy(dst.at[s, chunk], didx[slot][b])

        def run(h):
            for b in range(2):
                load_idx(0, b, b)

            def mbody(m, carry):
                c0 = m * 4
                for slot in range(2):
                    gds = []
                    for b in range(2):
                        @pl.when((m > 0) | (slot > 0))
                        def _(b=b, slot=slot):
                            pltpu.make_async_copy(
                                zrows.at[pl.ds(0, CHUNK)], rows[b],
                                ssem[1 - slot][b]).wait()
                        gds.append(pltpu.async_copy(
                            h.at[sidx[slot][b]], rows[b], gsem[b]))
                    for b in range(2):
                        nxt = c0 + slot * 2 + 2 + b

                        @pl.when(nxt < NCH)
                        def _(b=b, slot=slot, nxt=nxt):
                            load_idx(1 - slot, b, nxt)
                    for b in range(2):
                        gds[b].wait()
                        pltpu.async_copy(rows[b], shared.at[didx[slot][b]],
                                         ssem[slot][b], add=True)
                return carry

            lax.fori_loop(0, MEGA, mbody, 0)
            for b in range(2):
                pltpu.make_async_copy(zrows.at[pl.ds(0, CHUNK)], rows[b],
                                      ssem[1][b]).wait()

        @pl.when(c == 0)
        def _():
            run(h0)

        @pl.when(c == 1)
        def _():
            run(h1)

        plsc.subcore_barrier()

        r0 = s * RPT

        @pl.when(c == 0)
        def _():
            pltpu.sync_copy(shared.at[pl.ds(r0, RPT)], out0.at[pl.ds(r0, RPT)])

        @pl.when(c == 1)
        def _():
            pltpu.sync_copy(shared.at[pl.ds(r0, RPT)], out1.at[pl.ds(r0, RPT)])

    return agg_kernel


def _layer_body(eps_ref, h0, h1, a0, a1, A_ref, av_ref, B_ref, bv_ref, o0, o1):
    h = jnp.concatenate([h0[...], h1[...]], axis=1)
    agg = jnp.concatenate([a0[...], a1[...]], axis=1)
    z = (1.0 + eps_ref[0]) * h + agg
    t = jnp.dot(z, A_ref[...], preferred_element_type=jnp.float32) + av_ref[...]
    t = jnp.maximum(t, 0.0)
    o = jnp.dot(t, B_ref[...], preferred_element_type=jnp.float32) + bv_ref[...]
    o = jnp.maximum(o, 0.0)
    o0[...] = o[:, :HHALF]
    o1[...] = o[:, HHALF:]


def _make_layer():
    row = lambda i: (i, 0)
    full = lambda i: (0, 0)
    return pl.pallas_call(
        _layer_body,
        grid=(GRID,),
        in_specs=[
            pl.BlockSpec(memory_space=pltpu.SMEM),
            pl.BlockSpec((ROWBLK, HHALF), row),
            pl.BlockSpec((ROWBLK, HHALF), row),
            pl.BlockSpec((ROWBLK, HHALF), row),
            pl.BlockSpec((ROWBLK, HHALF), row),
            pl.BlockSpec((HID, 2 * HID), full),
            pl.BlockSpec((1, 2 * HID), full),
            pl.BlockSpec((2 * HID, HID), full),
            pl.BlockSpec((1, HID), full),
        ],
        out_specs=[
            pl.BlockSpec((ROWBLK, HHALF), row),
            pl.BlockSpec((ROWBLK, HHALF), row),
        ],
        out_shape=[
            jax.ShapeDtypeStruct((NPAD, HHALF), jnp.float32),
            jax.ShapeDtypeStruct((NPAD, HHALF), jnp.float32),
        ],
    )


def _pool_body(b_ref, h0, h1, out_ref):
    i = pl.program_id(0)
    h = jnp.concatenate([h0[...], h1[...]], axis=1)
    b = b_ref[0, 0, :]
    oh = (lax.broadcasted_iota(jnp.int32, (NSEG, ROWBLK), 0) == b[None, :])
    part = jnp.dot(oh.astype(jnp.float32), h, preferred_element_type=jnp.float32)

    @pl.when(i == 0)
    def _():
        out_ref[...] = part

    @pl.when(i > 0)
    def _():
        out_ref[...] += part


def _make_pool():
    row = lambda i: (i, 0)
    return pl.pallas_call(
        _pool_body,
        grid=(GRID,),
        in_specs=[
            pl.BlockSpec((1, 1, ROWBLK), lambda i: (i, 0, 0)),
            pl.BlockSpec((ROWBLK, HHALF), row),
            pl.BlockSpec((ROWBLK, HHALF), row),
        ],
        out_specs=pl.BlockSpec((NSEG, HID), lambda i: (0, 0)),
        out_shape=jax.ShapeDtypeStruct((NSEG, HID), jnp.float32),
    )


def _head_body(px, py, l1W, l1b, l2W, l2b, attw, attb, fW, fb, oW, ob, out_ref):
    def branch(p):
        q = jnp.dot(p[...], l1W[...], preferred_element_type=jnp.float32) + l1b[...]
        q = jnp.maximum(q, 0.0)
        return jnp.dot(q, l2W[...], preferred_element_type=jnp.float32) + l2b[...]

    xy = jnp.concatenate([branch(px), branch(py)], axis=1)
    logit = jnp.sum(xy * attw[...], axis=1, keepdims=True) + attb[0, 0]
    aw = jax.nn.sigmoid(logit)
    xy = aw * xy
    f = jnp.dot(xy, fW[...], preferred_element_type=jnp.float32) + fb[...]
    f = jnp.maximum(f, 0.0)
    lg = jnp.dot(f, oW[...], preferred_element_type=jnp.float32) + ob[...]
    m = jnp.max(lg, axis=1, keepdims=True)
    e = jnp.exp(lg - m)
    out_ref[...] = e / jnp.sum(e, axis=1, keepdims=True)


def _make_head():
    return pl.pallas_call(
        _head_body,
        out_shape=jax.ShapeDtypeStruct((NSEG, OUT), jnp.float32),
    )


_AGG = _make_agg()
_LAYER = _make_layer()
_POOL = _make_pool()
_HEAD = _make_head()


def kernel(x, y, edge_index_x, edge_index_y, batch_x, batch_y, Wb1, bb1, gb1, beb1, Wb2, bb2, epsb, gob, bob, Wp1, bp1, gp1, bep1, Wp2, bp2, epsp, gop, bop, lin1_W, lin1_b, bn1_g, bn1_b, lin2_W, lin2_b, att_W, att_b, fuse_W, fuse_b, out_W, out_b):
    inv = (1.0 + BN_EPS) ** -0.5

    def fold(W1, b1, g1, be1, W2, b2, go, bo):
        g1f = g1 * inv
        gof = go * inv
        A = W1 * g1f[:, None, :]
        av = (b1 * g1f + be1)[:, None, :]
        B = W2 * gof[:, None, :]
        bv = (b2 * gof + bo)[:, None, :]
        return A, av, B, bv

    Ab, avb, Bb, bvb = fold(Wb1, bb1, gb1, beb1, Wb2, bb2, gob, bob)
    Ap, avp, Bp, bvp = fold(Wp1, bp1, gp1, bep1, Wp2, bp2, gop, bop)

    def prep_nodes(v):
        vp = jnp.pad(v, ((0, NPAD - N), (0, 0)))
        return vp[:, :HHALF], vp[:, HHALF:]

    def prep_edges(ei):
        src, dst = lax.sort((ei[0], ei[1]), num_keys=1)
        src = jnp.pad(src, (0, EPAD - E))
        dst = jnp.pad(dst, (0, EPAD - E), constant_values=NPAD - 1)
        return src, dst.reshape(16, NCH, CHUNK)

    x0, x1 = prep_nodes(x)
    y0, y1 = prep_nodes(y)
    srcx, dstx = prep_edges(edge_index_x)
    srcy, dsty = prep_edges(edge_index_y)
    zrows = jnp.zeros((RPT, HHALF), jnp.float32)
    bxp = jnp.pad(batch_x, (0, NPAD - N), constant_values=-1).reshape(GRID, 1, ROWBLK)
    byp = jnp.pad(batch_y, (0, NPAD - N), constant_values=-1).reshape(GRID, 1, ROWBLK)

    hx0, hx1 = x0, x1
    hy0, hy1 = y0, y1
    for i in range(L):
        ax0, ax1 = _AGG(hx0, hx1, srcx, dstx, zrows)
        ay0, ay1 = _AGG(hy0, hy1, srcy, dsty, zrows)
        hx0, hx1 = _LAYER(jnp.reshape(epsb[i], (1,)), hx0, hx1, ax0, ax1,
                          Ab[i], avb[i], Bb[i], bvb[i])
        hy0, hy1 = _LAYER(jnp.reshape(epsp[i], (1,)), hy0, hy1, ay0, ay1,
                          Ap[i], avp[i], Bp[i], bvp[i])

    px = _POOL(bxp, hx0, hx1)
    py = _POOL(byp, hy0, hy1)

    g1f = bn1_g * inv
    lin1f = lin1_W * g1f[None, :]
    l1bf = (lin1_b * g1f + bn1_b)[None, :]

    return _HEAD(
        px, py, lin1f, l1bf, lin2_W, lin2_b[None, :],
        jnp.reshape(att_W, (1, 2 * HID)), jnp.reshape(att_b, (1, 1)),
        fuse_W, fuse_b[None, :], out_W, out_b[None, :],
    )


# speedup vs baseline: 1.1257x; 1.1257x over previous
import functools

import jax
import jax.numpy as jnp
from jax import lax
from jax.experimental import pallas as pl
from jax.experimental.pallas import tpu as pltpu
from jax.experimental.pallas import tpu_sc as plsc

N = 10000
HID = 256
HHALF = 128
E = 160000
NSEG = 64
L = 3
OUT = 64
BN_EPS = 1e-05

NPAD = 10240
EPAD = 163840
ROWBLK = 512
GRID = NPAD // ROWBLK
CHUNK = 128
EPG = EPAD // 16
NCH = EPG // CHUNK
RPT = NPAD // 16


def _make_agg():
    mesh = plsc.VectorSubcoreMesh(core_axis_name="c", subcore_axis_name="s")
    MEGA = NCH // 4

    @functools.partial(
        pl.kernel,
        out_type=[
            jax.ShapeDtypeStruct((NPAD, HHALF), jnp.float32),
            jax.ShapeDtypeStruct((NPAD, HHALF), jnp.float32),
        ],
        mesh=mesh,
        scratch_types=[
            [[pltpu.VMEM((CHUNK,), jnp.int32) for _ in range(2)] for _ in range(2)],
            [[pltpu.VMEM((CHUNK,), jnp.int32) for _ in range(2)] for _ in range(2)],
            [pltpu.VMEM((CHUNK, HHALF), jnp.float32) for _ in range(2)],
            [pltpu.SemaphoreType.DMA for _ in range(2)],
            [[pltpu.SemaphoreType.DMA for _ in range(2)] for _ in range(2)],
            pltpu.VMEM_SHARED((NPAD, HHALF), jnp.float32),
        ],
    )
    def agg_kernel(h0, h1, src, dst, zrows, out0, out1,
                   sidx, didx, rows, gsem, ssem, shared):
        c = lax.axis_index("c")
        s = lax.axis_index("s")
        pltpu.sync_copy(zrows, shared.at[pl.ds(s * RPT, RPT)])
        plsc.subcore_barrier()
        ebase = s * EPG

        def load_idx(slot, b, chunk):
            pltpu.sync_copy(src.at[pl.ds(ebase + chunk * CHUNK, CHUNK)],
                            sidx[slot][b])
            pltpu.sync_copy(dst.at[s, chunk], didx[slot][b])

        def run(h):
            for b in range(2):
                load_idx(0, b, b)

            def mbody(m, carry):
                c0 = m * 4
                for slot in range(2):
                    gds = []
                    for b in range(2):
                        @pl.when((m > 0) | (slot > 0))
                        def _(b=b, slot=slot):
                            pltpu.make_async_copy(
                                zrows.at[pl.ds(0, CHUNK)], rows[b],
                                ssem[1 - slot][b]).wait()
                        gds.append(pltpu.async_copy(
                            h.at[sidx[slot][b]], rows[b], gsem[b]))
                    for b in range(2):
                        nxt = c0 + slot * 2 + 2 + b

                        @pl.when(nxt < NCH)
                        def _(b=b, slot=slot, nxt=nxt):
                            load_idx(1 - slot, b, nxt)
                    for b in range(2):
                        gds[b].wait()
                        pltpu.async_copy(rows[b], shared.at[didx[slot][b]],
                                         ssem[slot][b], add=True)
                return carry

            lax.fori_loop(0, MEGA, mbody, 0)
            for b in range(2):
                pltpu.make_async_copy(zrows.at[pl.ds(0, CHUNK)], rows[b],
                                      ssem[1][b]).wait()

        @pl.when(c == 0)
        def _():
            run(h0)

        @pl.when(c == 1)
        def _():
            run(h1)

        plsc.subcore_barrier()

        r0 = s * RPT

        @pl.when(c == 0)
        def _():
            pltpu.sync_copy(shared.at[pl.ds(r0, RPT)], out0.at[pl.ds(r0, RPT)])

        @pl.when(c == 1)
        def _():
            pltpu.sync_copy(shared.at[pl.ds(r0, RPT)], out1.at[pl.ds(r0, RPT)])

    return agg_kernel


def _layer_body(eps_ref, h0, h1, a0, a1, A_ref, av_ref, B_ref, bv_ref, o0, o1):
    h = jnp.concatenate([h0[...], h1[...]], axis=1)
    agg = jnp.concatenate([a0[...], a1[...]], axis=1)
    z = (1.0 + eps_ref[0]) * h + agg
    t = jnp.dot(z, A_ref[...], preferred_element_type=jnp.float32) + av_ref[...]
    t = jnp.maximum(t, 0.0)
    o = jnp.dot(t, B_ref[...], preferred_element_type=jnp.float32) + bv_ref[...]
    o = jnp.maximum(o, 0.0)
    o0[...] = o[:, :HHALF]
    o1[...] = o[:, HHALF:]


def _make_layer():
    row = lambda i: (i, 0)
    full = lambda i: (0, 0)
    return pl.pallas_call(
        _layer_body,
        grid=(GRID,),
        in_specs=[
            pl.BlockSpec(memory_space=pltpu.SMEM),
            pl.BlockSpec((ROWBLK, HHALF), row),
            pl.BlockSpec((ROWBLK, HHALF), row),
            pl.BlockSpec((ROWBLK, HHALF), row),
            pl.BlockSpec((ROWBLK, HHALF), row),
            pl.BlockSpec((HID, 2 * HID), full),
            pl.BlockSpec((1, 2 * HID), full),
            pl.BlockSpec((2 * HID, HID), full),
            pl.BlockSpec((1, HID), full),
        ],
        out_specs=[
            pl.BlockSpec((ROWBLK, HHALF), row),
            pl.BlockSpec((ROWBLK, HHALF), row),
        ],
        out_shape=[
            jax.ShapeDtypeStruct((NPAD, HHALF), jnp.float32),
            jax.ShapeDtypeStruct((NPAD, HHALF), jnp.float32),
        ],
    )


def _pool_body(b_ref, h0, h1, out_ref):
    i = pl.program_id(0)
    h = jnp.concatenate([h0[...], h1[...]], axis=1)
    b = b_ref[0, 0, :]
    oh = (lax.broadcasted_iota(jnp.int32, (NSEG, ROWBLK), 0) == b[None, :])
    part = jnp.dot(oh.astype(jnp.float32), h, preferred_element_type=jnp.float32)

    @pl.when(i == 0)
    def _():
        out_ref[...] = part

    @pl.when(i > 0)
    def _():
        out_ref[...] += part


def _make_pool():
    row = lambda i: (i, 0)
    return pl.pallas_call(
        _pool_body,
        grid=(GRID,),
        in_specs=[
            pl.BlockSpec((1, 1, ROWBLK), lambda i: (i, 0, 0)),
            pl.BlockSpec((ROWBLK, HHALF), row),
            pl.BlockSpec((ROWBLK, HHALF), row),
        ],
        out_specs=pl.BlockSpec((NSEG, HID), lambda i: (0, 0)),
        out_shape=jax.ShapeDtypeStruct((NSEG, HID), jnp.float32),
    )


def _head_body(px, py, l1W, l1b, l2W, l2b, attw, attb, fW, fb, oW, ob, out_ref):
    def branch(p):
        q = jnp.dot(p[...], l1W[...], preferred_element_type=jnp.float32) + l1b[...]
        q = jnp.maximum(q, 0.0)
        return jnp.dot(q, l2W[...], preferred_element_type=jnp.float32) + l2b[...]

    xy = jnp.concatenate([branch(px), branch(py)], axis=1)
    logit = jnp.sum(xy * attw[...], axis=1, keepdims=True) + attb[0, 0]
    aw = jax.nn.sigmoid(logit)
    xy = aw * xy
    f = jnp.dot(xy, fW[...], preferred_element_type=jnp.float32) + fb[...]
    f = jnp.maximum(f, 0.0)
    lg = jnp.dot(f, oW[...], preferred_element_type=jnp.float32) + ob[...]
    m = jnp.max(lg, axis=1, keepdims=True)
    e = jnp.exp(lg - m)
    out_ref[...] = e / jnp.sum(e, axis=1, keepdims=True)


def _make_head():
    return pl.pallas_call(
        _head_body,
        out_shape=jax.ShapeDtypeStruct((NSEG, OUT), jnp.float32),
    )


_AGG = _make_agg()
_LAYER = _make_layer()
_POOL = _make_pool()
_HEAD = _make_head()


def kernel(x, y, edge_index_x, edge_index_y, batch_x, batch_y, Wb1, bb1, gb1, beb1, Wb2, bb2, epsb, gob, bob, Wp1, bp1, gp1, bep1, Wp2, bp2, epsp, gop, bop, lin1_W, lin1_b, bn1_g, bn1_b, lin2_W, lin2_b, att_W, att_b, fuse_W, fuse_b, out_W, out_b):
    inv = (1.0 + BN_EPS) ** -0.5

    def fold(W1, b1, g1, be1, W2, b2, go, bo):
        g1f = g1 * inv
        gof = go * inv
        A = W1 * g1f[:, None, :]
        av = (b1 * g1f + be1)[:, None, :]
        B = W2 * gof[:, None, :]
        bv = (b2 * gof + bo)[:, None, :]
        return A, av, B, bv

    Ab, avb, Bb, bvb = fold(Wb1, bb1, gb1, beb1, Wb2, bb2, gob, bob)
    Ap, avp, Bp, bvp = fold(Wp1, bp1, gp1, bep1, Wp2, bp2, gop, bop)

    def prep_nodes(v):
        vp = jnp.pad(v, ((0, NPAD - N), (0, 0)))
        return vp[:, :HHALF], vp[:, HHALF:]

    def prep_edges(ei):
        src = jnp.pad(ei[0], (0, EPAD - E))
        dst = jnp.pad(ei[1], (0, EPAD - E), constant_values=NPAD - 1)
        return src, dst.reshape(16, NCH, CHUNK)

    x0, x1 = prep_nodes(x)
    y0, y1 = prep_nodes(y)
    srcx, dstx = prep_edges(edge_index_x)
    srcy, dsty = prep_edges(edge_index_y)
    zrows = jnp.zeros((RPT, HHALF), jnp.float32)
    bxp = jnp.pad(batch_x, (0, NPAD - N), constant_values=-1).reshape(GRID, 1, ROWBLK)
    byp = jnp.pad(batch_y, (0, NPAD - N), constant_values=-1).reshape(GRID, 1, ROWBLK)

    hx0, hx1 = x0, x1
    hy0, hy1 = y0, y1
    for i in range(L):
        ax0, ax1 = _AGG(hx0, hx1, srcx, dstx, zrows)
        ay0, ay1 = _AGG(hy0, hy1, srcy, dsty, zrows)
        hx0, hx1 = _LAYER(jnp.reshape(epsb[i], (1,)), hx0, hx1, ax0, ax1,
                          Ab[i], avb[i], Bb[i], bvb[i])
        hy0, hy1 = _LAYER(jnp.reshape(epsp[i], (1,)), hy0, hy1, ay0, ay1,
                          Ap[i], avp[i], Bp[i], bvp[i])

    px = _POOL(bxp, hx0, hx1)
    py = _POOL(byp, hy0, hy1)

    g1f = bn1_g * inv
    lin1f = lin1_W * g1f[None, :]
    l1bf = (lin1_b * g1f + bn1_b)[None, :]

    return _HEAD(
        px, py, lin1f, l1bf, lin2_W, lin2_b[None, :],
        jnp.reshape(att_W, (1, 2 * HID)), jnp.reshape(att_b, (1, 1)),
        fuse_W, fuse_b[None, :], out_W, out_b[None, :],
    )


# speedup vs baseline: 1.1696x; 1.0390x over previous
import functools

import jax
import jax.numpy as jnp
from jax import lax
from jax.experimental import pallas as pl
from jax.experimental.pallas import tpu as pltpu
from jax.experimental.pallas import tpu_sc as plsc

N = 10000
HID = 256
HHALF = 128
E = 160000
NSEG = 64
L = 3
OUT = 64
BN_EPS = 1e-05

NPAD = 10240
EPAD = 163840
ROWBLK = 512
GRID = NPAD // ROWBLK
CHUNK = 128
EPG = EPAD // 16
NCH = EPG // CHUNK
RPT = NPAD // 16


def _make_agg():
    mesh = plsc.VectorSubcoreMesh(core_axis_name="c", subcore_axis_name="s")
    MEGA = NCH // 4

    @functools.partial(
        pl.kernel,
        out_type=[
            jax.ShapeDtypeStruct((NPAD, HHALF), jnp.float32),
            jax.ShapeDtypeStruct((NPAD, HHALF), jnp.float32),
        ],
        mesh=mesh,
        scratch_types=[
            pltpu.VMEM((EPG,), jnp.int32),
            [[pltpu.VMEM((CHUNK,), jnp.int32) for _ in range(2)] for _ in range(2)],
            [pltpu.VMEM((CHUNK, HHALF), jnp.float32) for _ in range(2)],
            [pltpu.SemaphoreType.DMA for _ in range(2)],
            [[pltpu.SemaphoreType.DMA for _ in range(2)] for _ in range(2)],
            pltpu.VMEM_SHARED((NPAD, HHALF), jnp.float32),
        ],
    )
    def agg_kernel(h0, h1, src, dst, zrows, out0, out1,
                   sidx, didx, rows, gsem, ssem, shared):
        c = lax.axis_index("c")
        s = lax.axis_index("s")
        pltpu.sync_copy(zrows, shared.at[pl.ds(s * RPT, RPT)])
        pltpu.sync_copy(src.at[pl.ds(s * EPG, EPG)], sidx)
        plsc.subcore_barrier()

        def load_idx(slot, b, chunk):
            pltpu.sync_copy(dst.at[s, chunk], didx[slot][b])

        def run(h):
            for b in range(2):
                load_idx(0, b, b)

            def mbody(m, carry):
                c0 = m * 4
                for slot in range(2):
                    gds = []
                    for b in range(2):
                        @pl.when((m > 0) | (slot > 0))
                        def _(b=b, slot=slot):
                            pltpu.make_async_copy(
                                zrows.at[pl.ds(0, CHUNK)], rows[b],
                                ssem[1 - slot][b]).wait()
                        gds.append(pltpu.async_copy(
                            h.at[sidx.at[pl.ds((c0 + slot * 2 + b) * CHUNK,
                                               CHUNK)]],
                            rows[b], gsem[b]))
                    for b in range(2):
                        nxt = c0 + slot * 2 + 2 + b

                        @pl.when(nxt < NCH)
                        def _(b=b, slot=slot, nxt=nxt):
                            load_idx(1 - slot, b, nxt)
                    for b in range(2):
                        gds[b].wait()
                        pltpu.async_copy(rows[b], shared.at[didx[slot][b]],
                                         ssem[slot][b], add=True)
                return carry

            lax.fori_loop(0, MEGA, mbody, 0)
            for b in range(2):
                pltpu.make_async_copy(zrows.at[pl.ds(0, CHUNK)], rows[b],
                                      ssem[1][b]).wait()

        @pl.when(c == 0)
        def _():
            run(h0)

        @pl.when(c == 1)
        def _():
            run(h1)

        plsc.subcore_barrier()

        r0 = s * RPT

        @pl.when(c == 0)
        def _():
            pltpu.sync_copy(shared.at[pl.ds(r0, RPT)], out0.at[pl.ds(r0, RPT)])

        @pl.when(c == 1)
        def _():
            pltpu.sync_copy(shared.at[pl.ds(r0, RPT)], out1.at[pl.ds(r0, RPT)])

    return agg_kernel


def _layer_body(eps_ref, h0, h1, a0, a1, A_ref, av_ref, B_ref, bv_ref, o0, o1):
    h = jnp.concatenate([h0[...], h1[...]], axis=1)
    agg = jnp.concatenate([a0[...], a1[...]], axis=1)
    z = (1.0 + eps_ref[0]) * h + agg
    t = jnp.dot(z, A_ref[...], preferred_element_type=jnp.float32) + av_ref[...]
    t = jnp.maximum(t, 0.0)
    o = jnp.dot(t, B_ref[...], preferred_element_type=jnp.float32) + bv_ref[...]
    o = jnp.maximum(o, 0.0)
    o0[...] = o[:, :HHALF]
    o1[...] = o[:, HHALF:]


def _make_layer():
    row = lambda i: (i, 0)
    full = lambda i: (0, 0)
    return pl.pallas_call(
        _layer_body,
        grid=(GRID,),
        in_specs=[
            pl.BlockSpec(memory_space=pltpu.SMEM),
            pl.BlockSpec((ROWBLK, HHALF), row),
            pl.BlockSpec((ROWBLK, HHALF), row),
            pl.BlockSpec((ROWBLK, HHALF), row),
            pl.BlockSpec((ROWBLK, HHALF), row),
            pl.BlockSpec((HID, 2 * HID), full),
            pl.BlockSpec((1, 2 * HID), full),
            pl.BlockSpec((2 * HID, HID), full),
            pl.BlockSpec((1, HID), full),
        ],
        out_specs=[
            pl.BlockSpec((ROWBLK, HHALF), row),
            pl.BlockSpec((ROWBLK, HHALF), row),
        ],
        out_shape=[
            jax.ShapeDtypeStruct((NPAD, HHALF), jnp.float32),
            jax.ShapeDtypeStruct((NPAD, HHALF), jnp.float32),
        ],
    )


def _pool_body(b_ref, h0, h1, out_ref):
    i = pl.program_id(0)
    h = jnp.concatenate([h0[...], h1[...]], axis=1)
    b = b_ref[0, 0, :]
    oh = (lax.broadcasted_iota(jnp.int32, (NSEG, ROWBLK), 0) == b[None, :])
    part = jnp.dot(oh.astype(jnp.float32), h, preferred_element_type=jnp.float32)

    @pl.when(i == 0)
    def _():
        out_ref[...] = part

    @pl.when(i > 0)
    def _():
        out_ref[...] += part


def _make_pool():
    row = lambda i: (i, 0)
    return pl.pallas_call(
        _pool_body,
        grid=(GRID,),
        in_specs=[
            pl.BlockSpec((1, 1, ROWBLK), lambda i: (i, 0, 0)),
            pl.BlockSpec((ROWBLK, HHALF), row),
            pl.BlockSpec((ROWBLK, HHALF), row),
        ],
        out_specs=pl.BlockSpec((NSEG, HID), lambda i: (0, 0)),
        out_shape=jax.ShapeDtypeStruct((NSEG, HID), jnp.float32),
    )


def _head_body(px, py, l1W, l1b, l2W, l2b, attw, attb, fW, fb, oW, ob, out_ref):
    def branch(p):
        q = jnp.dot(p[...], l1W[...], preferred_element_type=jnp.float32) + l1b[...]
        q = jnp.maximum(q, 0.0)
        return jnp.dot(q, l2W[...], preferred_element_type=jnp.float32) + l2b[...]

    xy = jnp.concatenate([branch(px), branch(py)], axis=1)
    logit = jnp.sum(xy * attw[...], axis=1, keepdims=True) + attb[0, 0]
    aw = jax.nn.sigmoid(logit)
    xy = aw * xy
    f = jnp.dot(xy, fW[...], preferred_element_type=jnp.float32) + fb[...]
    f = jnp.maximum(f, 0.0)
    lg = jnp.dot(f, oW[...], preferred_element_type=jnp.float32) + ob[...]
    m = jnp.max(lg, axis=1, keepdims=True)
    e = jnp.exp(lg - m)
    out_ref[...] = e / jnp.sum(e, axis=1, keepdims=True)


def _make_head():
    return pl.pallas_call(
        _head_body,
        out_shape=jax.ShapeDtypeStruct((NSEG, OUT), jnp.float32),
    )


_AGG = _make_agg()
_LAYER = _make_layer()
_POOL = _make_pool()
_HEAD = _make_head()


def kernel(x, y, edge_index_x, edge_index_y, batch_x, batch_y, Wb1, bb1, gb1, beb1, Wb2, bb2, epsb, gob, bob, Wp1, bp1, gp1, bep1, Wp2, bp2, epsp, gop, bop, lin1_W, lin1_b, bn1_g, bn1_b, lin2_W, lin2_b, att_W, att_b, fuse_W, fuse_b, out_W, out_b):
    inv = (1.0 + BN_EPS) ** -0.5

    def fold(W1, b1, g1, be1, W2, b2, go, bo):
        g1f = g1 * inv
        gof = go * inv
        A = W1 * g1f[:, None, :]
        av = (b1 * g1f + be1)[:, None, :]
        B = W2 * gof[:, None, :]
        bv = (b2 * gof + bo)[:, None, :]
        return A, av, B, bv

    Ab, avb, Bb, bvb = fold(Wb1, bb1, gb1, beb1, Wb2, bb2, gob, bob)
    Ap, avp, Bp, bvp = fold(Wp1, bp1, gp1, bep1, Wp2, bp2, gop, bop)

    def prep_nodes(v):
        vp = jnp.pad(v, ((0, NPAD - N), (0, 0)))
        return vp[:, :HHALF], vp[:, HHALF:]

    def prep_edges(ei):
        src = jnp.pad(ei[0], (0, EPAD - E))
        dst = jnp.pad(ei[1], (0, EPAD - E), constant_values=NPAD - 1)
        return src, dst.reshape(16, NCH, CHUNK)

    x0, x1 = prep_nodes(x)
    y0, y1 = prep_nodes(y)
    srcx, dstx = prep_edges(edge_index_x)
    srcy, dsty = prep_edges(edge_index_y)
    zrows = jnp.zeros((RPT, HHALF), jnp.float32)
    bxp = jnp.pad(batch_x, (0, NPAD - N), constant_values=-1).reshape(GRID, 1, ROWBLK)
    byp = jnp.pad(batch_y, (0, NPAD - N), constant_values=-1).reshape(GRID, 1, ROWBLK)

    hx0, hx1 = x0, x1
    hy0, hy1 = y0, y1
    for i in range(L):
        ax0, ax1 = _AGG(hx0, hx1, srcx, dstx, zrows)
        ay0, ay1 = _AGG(hy0, hy1, srcy, dsty, zrows)
        hx0, hx1 = _LAYER(jnp.reshape(epsb[i], (1,)), hx0, hx1, ax0, ax1,
                          Ab[i], avb[i], Bb[i], bvb[i])
        hy0, hy1 = _LAYER(jnp.reshape(epsp[i], (1,)), hy0, hy1, ay0, ay1,
                          Ap[i], avp[i], Bp[i], bvp[i])

    px = _POOL(bxp, hx0, hx1)
    py = _POOL(byp, hy0, hy1)

    g1f = bn1_g * inv
    lin1f = lin1_W * g1f[None, :]
    l1bf = (lin1_b * g1f + bn1_b)[None, :]

    return _HEAD(
        px, py, lin1f, l1bf, lin2_W, lin2_b[None, :],
        jnp.reshape(att_W, (1, 2 * HID)), jnp.reshape(att_b, (1, 1)),
        fuse_W, fuse_b[None, :], out_W, out_b[None, :],
    )


# speedup vs baseline: 1.1754x; 1.0050x over previous
import functools

import jax
import jax.numpy as jnp
from jax import lax
from jax.experimental import pallas as pl
from jax.experimental.pallas import tpu as pltpu
from jax.experimental.pallas import tpu_sc as plsc

N = 10000
HID = 256
HHALF = 128
E = 160000
NSEG = 64
L = 3
OUT = 64
BN_EPS = 1e-05

NPAD = 10240
EPAD = 163840
ROWBLK = 512
GRID = NPAD // ROWBLK
CHUNK = 128
EPG = EPAD // 16
NCH = EPG // CHUNK
RPT = NPAD // 16


def _make_agg():
    mesh = plsc.VectorSubcoreMesh(core_axis_name="c", subcore_axis_name="s")
    MEGA = NCH // 4

    @functools.partial(
        pl.kernel,
        out_type=[
            jax.ShapeDtypeStruct((NPAD, HHALF), jnp.float32),
            jax.ShapeDtypeStruct((NPAD, HHALF), jnp.float32),
        ],
        mesh=mesh,
        scratch_types=[
            pltpu.VMEM((EPG,), jnp.int32),
            [[pltpu.VMEM((CHUNK,), jnp.int32) for _ in range(2)] for _ in range(2)],
            [pltpu.VMEM((CHUNK, HHALF), jnp.float32) for _ in range(2)],
            [pltpu.SemaphoreType.DMA for _ in range(2)],
            [[pltpu.SemaphoreType.DMA for _ in range(2)] for _ in range(2)],
            [[pltpu.SemaphoreType.DMA for _ in range(2)] for _ in range(2)],
            pltpu.VMEM_SHARED((NPAD, HHALF), jnp.float32),
        ],
    )
    def agg_kernel(h0, h1, src, dst, zrows, out0, out1,
                   sidx, didx, rows, gsem, ssem, isem, shared):
        c = lax.axis_index("c")
        s = lax.axis_index("s")
        zd = pltpu.async_copy(zrows, shared.at[pl.ds(s * RPT, RPT)], gsem[0])
        sd = pltpu.async_copy(src.at[pl.ds(s * EPG, EPG)], sidx, gsem[1])
        zd.wait()
        sd.wait()
        plsc.subcore_barrier()

        def load_idx(slot, b, chunk):
            pltpu.async_copy(dst.at[s, chunk], didx[slot][b], isem[slot][b])

        def wait_idx(slot, b):
            pltpu.make_async_copy(dst.at[s, 0], didx[slot][b],
                                  isem[slot][b]).wait()

        def run(h):
            for b in range(2):
                load_idx(0, b, b)
                wait_idx(0, b)

            def mbody(m, carry):
                c0 = m * 4
                for slot in range(2):
                    gds = []
                    for b in range(2):
                        @pl.when((m > 0) | (slot > 0))
                        def _(b=b, slot=slot):
                            pltpu.make_async_copy(
                                zrows.at[pl.ds(0, CHUNK)], rows[b],
                                ssem[1 - slot][b]).wait()
                        gds.append(pltpu.async_copy(
                            h.at[sidx.at[pl.ds((c0 + slot * 2 + b) * CHUNK,
                                               CHUNK)]],
                            rows[b], gsem[b]))
                    for b in range(2):
                        nxt = c0 + slot * 2 + 2 + b

                        @pl.when(nxt < NCH)
                        def _(b=b, slot=slot, nxt=nxt):
                            load_idx(1 - slot, b, nxt)
                    for b in range(2):
                        gds[b].wait()
                        @pl.when((m > 0) | (slot > 0))
                        def _(b=b, slot=slot):
                            wait_idx(slot, b)
                        pltpu.async_copy(rows[b], shared.at[didx[slot][b]],
                                         ssem[slot][b], add=True)
                return carry

            lax.fori_loop(0, MEGA, mbody, 0)
            for b in range(2):
                pltpu.make_async_copy(zrows.at[pl.ds(0, CHUNK)], rows[b],
                                      ssem[1][b]).wait()

        @pl.when(c == 0)
        def _():
            run(h0)

        @pl.when(c == 1)
        def _():
            run(h1)

        plsc.subcore_barrier()

        r0 = s * RPT

        @pl.when(c == 0)
        def _():
            pltpu.sync_copy(shared.at[pl.ds(r0, RPT)], out0.at[pl.ds(r0, RPT)])

        @pl.when(c == 1)
        def _():
            pltpu.sync_copy(shared.at[pl.ds(r0, RPT)], out1.at[pl.ds(r0, RPT)])

    return agg_kernel


def _layer_body(eps_ref, h0, h1, a0, a1, A_ref, av_ref, B_ref, bv_ref, o0, o1):
    h = jnp.concatenate([h0[...], h1[...]], axis=1)
    agg = jnp.concatenate([a0[...], a1[...]], axis=1)
    z = (1.0 + eps_ref[0]) * h + agg
    t = jnp.dot(z, A_ref[...], preferred_element_type=jnp.float32) + av_ref[...]
    t = jnp.maximum(t, 0.0)
    o = jnp.dot(t, B_ref[...], preferred_element_type=jnp.float32) + bv_ref[...]
    o = jnp.maximum(o, 0.0)
    o0[...] = o[:, :HHALF]
    o1[...] = o[:, HHALF:]


def _make_layer():
    row = lambda i: (i, 0)
    full = lambda i: (0, 0)
    return pl.pallas_call(
        _layer_body,
        grid=(GRID,),
        in_specs=[
            pl.BlockSpec(memory_space=pltpu.SMEM),
            pl.BlockSpec((ROWBLK, HHALF), row),
            pl.BlockSpec((ROWBLK, HHALF), row),
            pl.BlockSpec((ROWBLK, HHALF), row),
            pl.BlockSpec((ROWBLK, HHALF), row),
            pl.BlockSpec((HID, 2 * HID), full),
            pl.BlockSpec((1, 2 * HID), full),
            pl.BlockSpec((2 * HID, HID), full),
            pl.BlockSpec((1, HID), full),
        ],
        out_specs=[
            pl.BlockSpec((ROWBLK, HHALF), row),
            pl.BlockSpec((ROWBLK, HHALF), row),
        ],
        out_shape=[
            jax.ShapeDtypeStruct((NPAD, HHALF), jnp.float32),
            jax.ShapeDtypeStruct((NPAD, HHALF), jnp.float32),
        ],
    )


def _pool_body(b_ref, h0, h1, out_ref):
    i = pl.program_id(0)
    h = jnp.concatenate([h0[...], h1[...]], axis=1)
    b = b_ref[0, 0, :]
    oh = (lax.broadcasted_iota(jnp.int32, (NSEG, ROWBLK), 0) == b[None, :])
    part = jnp.dot(oh.astype(jnp.float32), h, preferred_element_type=jnp.float32)

    @pl.when(i == 0)
    def _():
        out_ref[...] = part

    @pl.when(i > 0)
    def _():
        out_ref[...] += part


def _make_pool():
    row = lambda i: (i, 0)
    return pl.pallas_call(
        _pool_body,
        grid=(GRID,),
        in_specs=[
            pl.BlockSpec((1, 1, ROWBLK), lambda i: (i, 0, 0)),
            pl.BlockSpec((ROWBLK, HHALF), row),
            pl.BlockSpec((ROWBLK, HHALF), row),
        ],
        out_specs=pl.BlockSpec((NSEG, HID), lambda i: (0, 0)),
        out_shape=jax.ShapeDtypeStruct((NSEG, HID), jnp.float32),
    )


def _head_body(px, py, l1W, l1b, l2W, l2b, attw, attb, fW, fb, oW, ob, out_ref):
    def branch(p):
        q = jnp.dot(p[...], l1W[...], preferred_element_type=jnp.float32) + l1b[...]
        q = jnp.maximum(q, 0.0)
        return jnp.dot(q, l2W[...], preferred_element_type=jnp.float32) + l2b[...]

    xy = jnp.concatenate([branch(px), branch(py)], axis=1)
    logit = jnp.sum(xy * attw[...], axis=1, keepdims=True) + attb[0, 0]
    aw = jax.nn.sigmoid(logit)
    xy = aw * xy
    f = jnp.dot(xy, fW[...], preferred_element_type=jnp.float32) + fb[...]
    f = jnp.maximum(f, 0.0)
    lg = jnp.dot(f, oW[...], preferred_element_type=jnp.float32) + ob[...]
    m = jnp.max(lg, axis=1, keepdims=True)
    e = jnp.exp(lg - m)
    out_ref[...] = e / jnp.sum(e, axis=1, keepdims=True)


def _make_head():
    return pl.pallas_call(
        _head_body,
        out_shape=jax.ShapeDtypeStruct((NSEG, OUT), jnp.float32),
    )


_AGG = _make_agg()
_LAYER = _make_layer()
_POOL = _make_pool()
_HEAD = _make_head()


def kernel(x, y, edge_index_x, edge_index_y, batch_x, batch_y, Wb1, bb1, gb1, beb1, Wb2, bb2, epsb, gob, bob, Wp1, bp1, gp1, bep1, Wp2, bp2, epsp, gop, bop, lin1_W, lin1_b, bn1_g, bn1_b, lin2_W, lin2_b, att_W, att_b, fuse_W, fuse_b, out_W, out_b):
    inv = (1.0 + BN_EPS) ** -0.5

    def fold(W1, b1, g1, be1, W2, b2, go, bo):
        g1f = g1 * inv
        gof = go * inv
        A = W1 * g1f[:, None, :]
        av = (b1 * g1f + be1)[:, None, :]
        B = W2 * gof[:, None, :]
        bv = (b2 * gof + bo)[:, None, :]
        return A, av, B, bv

    Ab, avb, Bb, bvb = fold(Wb1, bb1, gb1, beb1, Wb2, bb2, gob, bob)
    Ap, avp, Bp, bvp = fold(Wp1, bp1, gp1, bep1, Wp2, bp2, gop, bop)

    def prep_nodes(v):
        vp = jnp.pad(v, ((0, NPAD - N), (0, 0)))
        return vp[:, :HHALF], vp[:, HHALF:]

    def prep_edges(ei):
        src = jnp.pad(ei[0], (0, EPAD - E))
        dst = jnp.pad(ei[1], (0, EPAD - E), constant_values=NPAD - 1)
        return src, dst.reshape(16, NCH, CHUNK)

    x0, x1 = prep_nodes(x)
    y0, y1 = prep_nodes(y)
    srcx, dstx = prep_edges(edge_index_x)
    srcy, dsty = prep_edges(edge_index_y)
    zrows = jnp.zeros((RPT, HHALF), jnp.float32)
    bxp = jnp.pad(batch_x, (0, NPAD - N), constant_values=-1).reshape(GRID, 1, ROWBLK)
    byp = jnp.pad(batch_y, (0, NPAD - N), constant_values=-1).reshape(GRID, 1, ROWBLK)

    hx0, hx1 = x0, x1
    hy0, hy1 = y0, y1
    for i in range(L):
        ax0, ax1 = _AGG(hx0, hx1, srcx, dstx, zrows)
        ay0, ay1 = _AGG(hy0, hy1, srcy, dsty, zrows)
        hx0, hx1 = _LAYER(jnp.reshape(epsb[i], (1,)), hx0, hx1, ax0, ax1,
                          Ab[i], avb[i], Bb[i], bvb[i])
        hy0, hy1 = _LAYER(jnp.reshape(epsp[i], (1,)), hy0, hy1, ay0, ay1,
                          Ap[i], avp[i], Bp[i], bvp[i])

    px = _POOL(bxp, hx0, hx1)
    py = _POOL(byp, hy0, hy1)

    g1f = bn1_g * inv
    lin1f = lin1_W * g1f[None, :]
    l1bf = (lin1_b * g1f + bn1_b)[None, :]

    return _HEAD(
        px, py, lin1f, l1bf, lin2_W, lin2_b[None, :],
        jnp.reshape(att_W, (1, 2 * HID)), jnp.reshape(att_b, (1, 1)),
        fuse_W, fuse_b[None, :], out_W, out_b[None, :],
    )
